# baseline (speedup 1.0000x reference)
import functools

import jax
import jax.numpy as jnp
from jax import lax
from jax.experimental import pallas as pl
from jax.experimental.pallas import tpu as pltpu
from jax.experimental.pallas import tpu_sc as plsc

B = 16384
D = 64
L = 16
NC = 2
NS = 16
NW = NC * NS
BPW = B // NW
WIN = 32
NWIN = BPW // WIN

_mesh = plsc.VectorSubcoreMesh(core_axis_name="c", subcore_axis_name="s")


@functools.partial(
    pl.kernel,
    mesh=_mesh,
    compiler_params=pltpu.CompilerParams(needs_layout_passes=False),
    out_type=jax.ShapeDtypeStruct((B,), jnp.float32),
    scratch_types=[
        pltpu.VMEM((BPW,), jnp.int32),
        pltpu.VMEM((BPW,), jnp.int32),
        pltpu.VMEM((WIN, 8, D), jnp.float32),
        pltpu.VMEM((WIN, 8, D), jnp.float32),
        pltpu.VMEM((D,), jnp.float32),
        pltpu.VMEM((L,), jnp.float32),
        pltpu.VMEM((BPW,), jnp.float32),
        pltpu.SemaphoreType.DMA,
        pltpu.SemaphoreType.DMA,
    ],
)
def _sc_kernel(uid_hbm, iid_hbm, ut_hbm, it_hbm, w_hbm, b_hbm, out_hbm,
               uidx_v, iidx_v, ublk_v, iblk_v, w_v, b_v, out_v,
               usem, isem):
    wid = lax.axis_index("s") * NC + lax.axis_index("c")
    base = wid * BPW

    pltpu.sync_copy(uid_hbm.at[pl.ds(base, BPW)], uidx_v)
    pltpu.sync_copy(iid_hbm.at[pl.ds(base, BPW)], iidx_v)
    pltpu.sync_copy(w_hbm, w_v)
    pltpu.sync_copy(b_hbm, b_v)

    wvecs = [w_v[pl.ds(j * L, L)] for j in range(D // L)]
    bvec = b_v[...]
    liota = lax.iota(jnp.int32, L)

    def win_body(wi, carry):
        r0 = wi * WIN
        uvecs = [uidx_v[pl.ds(r0 + q * L, L)] for q in range(WIN // L)]
        ivecs = [iidx_v[pl.ds(r0 + q * L, L)] for q in range(WIN // L)]
        copies = []
        for k in range(WIN):
            u = uvecs[k // L][k % L]
            i = ivecs[k // L][k % L]
            ua = pl.multiple_of(u - (u % 8), 8)
            ia = pl.multiple_of(i - (i % 8), 8)
            copies.append(pltpu.async_copy(
                ut_hbm.at[pl.ds(ua, 8)], ublk_v.at[k], usem))
            copies.append(pltpu.async_copy(
                it_hbm.at[pl.ds(ia, 8)], iblk_v.at[k], isem))
        for cp in copies:
            cp.wait()

        for q in range(WIN // L):
            acc = bvec
            for rr in range(L):
                k = q * L + rr
                ru = uvecs[q][rr] % 8
                ri = ivecs[q][rr] % 8
                s = None
                for j in range(D // L):
                    t = (ublk_v[k, ru, pl.ds(j * L, L)]
                         * iblk_v[k, ri, pl.ds(j * L, L)] * wvecs[j])
                    s = t if s is None else s + t
                acc = jnp.where(liota == rr, acc + jnp.sum(s), acc)
            out_v[pl.ds(r0 + q * L, L)] = acc
        return carry

    lax.fori_loop(0, NWIN, win_body, 0, unroll=False)

    pltpu.sync_copy(out_v, out_hbm.at[pl.ds(base, BPW)])


def kernel(user_ids, item_ids, user_table, item_table, fc_w, fc_b):
    w = fc_w.reshape(D)
    b = jnp.broadcast_to(fc_b.reshape(1), (L,))
    return _sc_kernel(user_ids, item_ids, user_table, item_table, w, b)
